# baseline (speedup 1.0000x reference)
import functools
import jax
import jax.numpy as jnp
from jax import lax
from jax.experimental import pallas as pl
from jax.experimental.pallas import tpu as pltpu
from jax.experimental.pallas import tpu_sc as plsc

_B = 4
_K = 4096
_N = 2048

_NSC = 512
_NTC = _N - _NSC
_TCOL = 512

_NC = 2
_NS = 16
_NW = _NC * _NS
_CW = 128
_NCHUNKCOL = _NSC // _CW
_KH = _K // 2
_RC = 128
_NCHUNK = _KH // _RC
_NU = 4


def _tc_body(x_ref, o_ref):
    v = x_ref[0]
    mn = jnp.min(v, axis=0, keepdims=True)
    rows = jax.lax.broadcasted_iota(jnp.int32, v.shape, 0)
    big = jnp.int32(2**30)
    idx = jnp.min(jnp.where(v == mn, rows, big), axis=0)
    o_ref[0, 0] = idx


def _sc_argmin(x_hbm, out_hbm, buf, amin, aidx, pmin, pidx, sm, si, sem0, sem1):
    c = lax.axis_index("c")
    s = lax.axis_index("s")
    rowhalf = s % 2
    unit = s // 2
    b = c * 2 + unit // _NCHUNKCOL
    col0 = _NTC + (unit % _NCHUNKCOL) * _CW
    row0 = rowhalf * _KH

    for g in range(_CW // 16):
        amin[pl.ds(g * 16, 16)] = jnp.full((16,), jnp.inf, jnp.float32)
        aidx[pl.ds(g * 16, 16)] = jnp.zeros((16,), jnp.int32)

    def chunk_src(i):
        return x_hbm.at[b, pl.ds(row0 + i * _RC, _RC), pl.ds(col0, _CW)]

    pltpu.async_copy(chunk_src(0), buf.at[0], sem0)
    pltpu.async_copy(chunk_src(1), buf.at[1], sem1)

    def process(i, slot, sem):
        pltpu.make_async_copy(chunk_src(i), buf.at[slot], sem).wait()
        base = row0 + i * _RC

        def cg_body(g, carry):
            c0 = g * (16 * _NU)
            rm = [amin[pl.ds(c0 + u * 16, 16)] for u in range(_NU)]
            ri = [aidx[pl.ds(c0 + u * 16, 16)] for u in range(_NU)]
            for r in range(_RC):
                vr = jnp.full((16,), base + r, jnp.int32)
                for u in range(_NU):
                    v = buf[slot, r, pl.ds(c0 + u * 16, 16)]
                    m = v < rm[u]
                    rm[u] = jnp.where(m, v, rm[u])
                    ri[u] = jnp.where(m, vr, ri[u])
            for u in range(_NU):
                amin[pl.ds(c0 + u * 16, 16)] = rm[u]
                aidx[pl.ds(c0 + u * 16, 16)] = ri[u]
            return carry

        lax.fori_loop(0, _CW // (16 * _NU), cg_body, 0)

        nxt = i + 2

        @pl.when(nxt < _NCHUNK)
        def _():
            pltpu.async_copy(chunk_src(nxt), buf.at[slot], sem)

    def loop_body(j, carry):
        process(2 * j, 0, sem0)
        process(2 * j + 1, 1, sem1)
        return carry

    lax.fori_loop(0, _NCHUNK // 2, loop_body, 0)

    pltpu.sync_copy(amin, sm.at[s])
    pltpu.sync_copy(aidx, si.at[s])
    plsc.subcore_barrier()

    @pl.when(rowhalf == 0)
    def _():
        pltpu.sync_copy(sm.at[s + 1], pmin)
        pltpu.sync_copy(si.at[s + 1], pidx)
        for g in range(_CW // 16):
            sl = pl.ds(g * 16, 16)
            m = pmin[sl] < amin[sl]
            aidx[sl] = jnp.where(m, pidx[sl], aidx[sl])
        pltpu.sync_copy(aidx, out_hbm.at[b, pl.ds(col0 - _NTC, _CW)])


_sc_call = functools.partial(
    pl.kernel,
    out_type=jax.ShapeDtypeStruct((_B, _NSC), jnp.int32),
    mesh=plsc.VectorSubcoreMesh(core_axis_name="c", subcore_axis_name="s"),
    scratch_types=[
        pltpu.VMEM((2, _RC, _CW), jnp.float32),
        pltpu.VMEM((_CW,), jnp.float32),
        pltpu.VMEM((_CW,), jnp.int32),
        pltpu.VMEM((_CW,), jnp.float32),
        pltpu.VMEM((_CW,), jnp.int32),
        pltpu.VMEM_SHARED((_NS, _CW), jnp.float32),
        pltpu.VMEM_SHARED((_NS, _CW), jnp.int32),
        pltpu.SemaphoreType.DMA,
        pltpu.SemaphoreType.DMA,
    ],
)(_sc_argmin)


def kernel(x):
    b, k, n = x.shape
    sc_out = _sc_call(x)
    tc_out = pl.pallas_call(
        _tc_body,
        grid=(b, _NTC // _TCOL),
        in_specs=[pl.BlockSpec((1, k, _TCOL), lambda i, j: (i, 0, j))],
        out_specs=pl.BlockSpec((1, 1, _TCOL), lambda i, j: (i, 0, j)),
        out_shape=jax.ShapeDtypeStruct((b, 1, _NTC), jnp.int32),
    )(x)
    out = jnp.concatenate([tc_out.reshape(b, _NTC), sc_out], axis=1)
    return out.astype(jnp.int64)


# speedup vs baseline: 1.3009x; 1.3009x over previous
import jax
import jax.numpy as jnp
from jax.experimental import pallas as pl


_COLS = 512


def _argmin_body(x_ref, o_ref):
    v = x_ref[0]
    mn = jnp.min(v, axis=0, keepdims=True)
    rows = jax.lax.broadcasted_iota(jnp.int32, v.shape, 0)
    big = jnp.int32(2**30)
    idx = jnp.min(jnp.where(v == mn, rows, big), axis=0)
    o_ref[0, 0] = idx


def kernel(x):
    b, k, n = x.shape
    grid = (b, n // _COLS)
    out = pl.pallas_call(
        _argmin_body,
        grid=grid,
        in_specs=[pl.BlockSpec((1, k, _COLS), lambda i, j: (i, 0, j))],
        out_specs=pl.BlockSpec((1, 1, _COLS), lambda i, j: (i, 0, j)),
        out_shape=jax.ShapeDtypeStruct((b, 1, n), jnp.int32),
    )(x)
    return out.reshape(b, n).astype(jnp.int64)


# speedup vs baseline: 1.4001x; 1.0762x over previous
import jax
import jax.numpy as jnp
from jax.experimental import pallas as pl


_COLS = 1024


def _argmin_body(x_ref, o_ref):
    v = x_ref[0]
    mn = jnp.min(v, axis=0, keepdims=True)
    rows = jax.lax.broadcasted_iota(jnp.int32, v.shape, 0).astype(jnp.float32)
    big = jnp.float32(2**30)
    idx = jnp.min(jnp.where(v == mn, rows, big), axis=0)
    o_ref[0, 0] = idx.astype(jnp.int32)


def kernel(x):
    b, k, n = x.shape
    grid = (b, n // _COLS)
    out = pl.pallas_call(
        _argmin_body,
        grid=grid,
        in_specs=[pl.BlockSpec((1, k, _COLS), lambda i, j: (i, 0, j))],
        out_specs=pl.BlockSpec((1, 1, _COLS), lambda i, j: (i, 0, j)),
        out_shape=jax.ShapeDtypeStruct((b, 1, n), jnp.int32),
    )(x)
    return out.reshape(b, n).astype(jnp.int64)


# speedup vs baseline: 1.4256x; 1.0183x over previous
import jax
import jax.numpy as jnp
from jax.experimental import pallas as pl


_COLS = 1024


def _argmin_body(x_ref, o_ref):
    v = x_ref[0]
    k = v.shape[0]
    mn = jnp.min(v, axis=0, keepdims=True)
    big = jnp.int32(2**20)
    acc = jnp.full((8, v.shape[1]), big, jnp.int32)
    for s in range(k // 8 - 1, -1, -1):
        m = v[s * 8:(s + 1) * 8] == mn
        acc = jnp.where(m, jnp.int32(s), acc)
    rows = acc * 8 + jax.lax.broadcasted_iota(jnp.int32, acc.shape, 0)
    o_ref[0, 0] = jnp.min(rows, axis=0)


def kernel(x):
    b, k, n = x.shape
    grid = (b, n // _COLS)
    out = pl.pallas_call(
        _argmin_body,
        grid=grid,
        in_specs=[pl.BlockSpec((1, k, _COLS), lambda i, j: (i, 0, j))],
        out_specs=pl.BlockSpec((1, 1, _COLS), lambda i, j: (i, 0, j)),
        out_shape=jax.ShapeDtypeStruct((b, 1, n), jnp.int32),
    )(x)
    return out.reshape(b, n).astype(jnp.int64)
